# baseline (speedup 1.0000x reference)
import functools

import jax
import jax.numpy as jnp
from jax import lax
from jax.experimental import pallas as pl
from jax.experimental.pallas import tpu as pltpu
from jax.experimental.pallas import tpu_sc as plsc

N = 10000
E = 320000
NS, NV = 128, 16
ES, EV = 32, 1

HW = 128
BE = 2000
NSC = 2
NSUB = 16
EPW = E // NSUB
CH = 80
NCH = EPW // CH
NPAD = 10240
RPW = NPAD // NSUB

_HI = lax.Precision.HIGHEST


def _mm(a, b_t, prec=lax.Precision.HIGHEST):
    return lax.dot_general(a, b_t, (((1,), (1,)), ((), ())),
                           precision=prec, preferred_element_type=jnp.float32)


def _edge_kernel(es_ref, ev_ref, ws_ref, bs_ref, wv_ref, bv_ref, r_ref,
                 out_ref):
    es = es_ref[...]
    hi3 = lax.Precision.DEFAULT
    s_lin = _mm(es, ws_ref[...], hi3) + bs_ref[...]
    s_out = s_lin * jax.nn.sigmoid(s_lin)
    v_lin = _mm(ev_ref[...], wv_ref[...], hi3) + bv_ref[...]
    gate = jax.nn.sigmoid(s_out[:, :NV])
    gate48 = _mm(gate, r_ref[...], hi3)
    v_out = v_lin * gate48
    ones = jnp.ones((es.shape[0], 1), jnp.float32)
    pad = jnp.zeros((es.shape[0], HW - 3 * NV - 1), jnp.float32)
    out_ref[0] = s_out
    out_ref[1] = jnp.concatenate([v_out, ones, pad], axis=1)


def _edge_stage(edge_s, edge_v3, Ws_e, bs_e, Wv_e, bv_e, r48):
    grid = (E // BE,)
    return pl.pallas_call(
        _edge_kernel,
        grid=grid,
        in_specs=[
            pl.BlockSpec((BE, ES), lambda i: (i, 0)),
            pl.BlockSpec((BE, 3), lambda i: (i, 0)),
            pl.BlockSpec((NS, ES), lambda i: (0, 0)),
            pl.BlockSpec((1, NS), lambda i: (0, 0)),
            pl.BlockSpec((3 * NV, 3), lambda i: (0, 0)),
            pl.BlockSpec((1, 3 * NV), lambda i: (0, 0)),
            pl.BlockSpec((3 * NV, NV), lambda i: (0, 0)),
        ],
        out_specs=pl.BlockSpec((NSC, BE, HW), lambda i: (0, i, 0)),
        out_shape=jax.ShapeDtypeStruct((NSC, E, HW), jnp.float32),
    )(edge_s, edge_v3, Ws_e, bs_e.reshape(1, NS), Wv_e,
      bv_e.reshape(1, 3 * NV), r48)


def _scatter_body(msg_hbm, dst_hbm, out_hbm, idx_v, msg_v, zero_v, acc_sh):
    c = lax.axis_index("c")
    s = lax.axis_index("s")
    ebase = s * EPW

    def zrow(r, carry):
        for g in range(HW // 16):
            zero_v[r, pl.ds(g * 16, 16)] = jnp.zeros((16,), jnp.float32)
        return carry
    lax.fori_loop(0, zero_v.shape[0], zrow, 0)
    zr = zero_v.shape[0]
    for i in range(RPW // zr):
        pltpu.sync_copy(zero_v, acc_sh.at[pl.ds(s * RPW + i * zr, zr)])
    plsc.subcore_barrier()

    def chunk(i, carry):
        e0 = pl.multiple_of(ebase + i * CH, 8)
        pltpu.sync_copy(dst_hbm.at[pl.ds(e0, CH)], idx_v)
        pltpu.sync_copy(msg_hbm.at[c, pl.ds(e0, CH)], msg_v)
        pltpu.sync_copy(msg_v, acc_sh.at[idx_v], add=True)
        return carry
    lax.fori_loop(0, NCH, chunk, 0)
    plsc.subcore_barrier()

    pltpu.sync_copy(acc_sh.at[pl.ds(s * RPW, RPW)],
                    out_hbm.at[c, pl.ds(s * RPW, RPW)])


def _scatter_stage(msg, dst):
    mesh = plsc.VectorSubcoreMesh(core_axis_name="c", subcore_axis_name="s")
    f = pl.kernel(
        _scatter_body,
        out_type=jax.ShapeDtypeStruct((NSC, NPAD, HW), jnp.float32),
        mesh=mesh,
        scratch_types=[
            pltpu.VMEM((CH,), jnp.int32),
            pltpu.VMEM((CH, HW), jnp.float32),
            pltpu.VMEM((128, HW), jnp.float32),
            pltpu.VMEM_SHARED((NPAD, HW), jnp.float32),
        ],
    )
    return f(msg, dst)


def _node_kernel(p0_ref, p1_ref, ns_ref, nv_ref, lng_ref, lnb_ref,
                 wsn_ref, bsn_ref, wvn_ref, bvn_ref,
                 wrs_ref, brs_ref, wrv_ref, brv_ref, r_ref,
                 so_ref, vo_ref):
    p1 = p1_ref[...]
    cnt = p1[:, 3 * NV:3 * NV + 1]
    denom = jnp.maximum(cnt, 1.0)
    s_agg = p0_ref[...] / denom
    v_agg = p1[:, :3 * NV] / denom

    ns = ns_ref[...]
    mu = jnp.mean(ns, axis=1, keepdims=True)
    var = jnp.mean((ns - mu) ** 2, axis=1, keepdims=True)
    s = (ns - mu) / jnp.sqrt(var + 1e-5) * lng_ref[...] + lnb_ref[...]

    nv = nv_ref[...]
    vn = jnp.sqrt(jnp.sum(nv * nv, axis=1, keepdims=True) / NV + 1e-8)
    v = nv / vn

    hs_in = s + s_agg
    hv_in = v + v_agg
    s_lin = _mm(hs_in, wsn_ref[...]) + bsn_ref[...]
    h_s = s_lin * jax.nn.sigmoid(s_lin)
    v_lin = _mm(hv_in, wvn_ref[...]) + bvn_ref[...]
    gate48 = _mm(jax.nn.sigmoid(h_s[:, :NV]), r_ref[...])
    h_v = v_lin * gate48

    so_ref[...] = h_s + _mm(s, wrs_ref[...]) + brs_ref[...]
    vo_ref[...] = h_v + _mm(v, wrv_ref[...]) + brv_ref[...]


BN = 2000


def _node_stage(partials, node_s, node_v48, ln_g, ln_b, Ws_n, bs_n, Wv_n,
                bv_n, Wr_s, br_s, Wr_v, br_v, r48):
    blk = lambda shape: pl.BlockSpec(shape, lambda i: (i,) + (0,) * (len(shape) - 1))
    fix = lambda shape: pl.BlockSpec(shape, lambda i: (0,) * len(shape))
    return pl.pallas_call(
        _node_kernel,
        grid=(N // BN,),
        in_specs=[
            blk((BN, HW)), blk((BN, HW)), blk((BN, NS)), blk((BN, 3 * NV)),
            fix((1, NS)), fix((1, NS)),
            fix((NS, NS)), fix((1, NS)),
            fix((3 * NV, 3 * NV)), fix((1, 3 * NV)),
            fix((NS, NS)), fix((1, NS)),
            fix((3 * NV, 3 * NV)), fix((1, 3 * NV)),
            fix((3 * NV, NV)),
        ],
        out_specs=[blk((BN, NS)), blk((BN, 3 * NV))],
        out_shape=[jax.ShapeDtypeStruct((N, NS), jnp.float32),
                   jax.ShapeDtypeStruct((N, 3 * NV), jnp.float32)],
    )(partials[0], partials[1], node_s, node_v48,
      ln_g.reshape(1, NS), ln_b.reshape(1, NS),
      Ws_n, bs_n.reshape(1, NS), Wv_n, bv_n.reshape(1, 3 * NV),
      Wr_s, br_s.reshape(1, NS), Wr_v, br_v.reshape(1, 3 * NV), r48)


def kernel(node_s, node_v, edge_s, edge_v, ln_g, ln_b, Ws_e, bs_e, Wv_e,
           bv_e, Ws_n, bs_n, Wv_n, bv_n, Wr_s, br_s, Wr_v, br_v, edge_index):
    edge_v3 = edge_v.reshape(E, 3 * EV)
    node_v48 = node_v.reshape(N, 3 * NV)
    dst = edge_index[1].astype(jnp.int32)
    r48 = jnp.kron(jnp.eye(NV, dtype=jnp.float32),
                   jnp.ones((3, 1), jnp.float32))

    msg = _edge_stage(edge_s, edge_v3, Ws_e, bs_e, Wv_e, bv_e, r48)
    partials = _scatter_stage(msg, dst)[:, :N, :]
    s_out, v_out48 = _node_stage(partials, node_s, node_v48, ln_g, ln_b,
                                 Ws_n, bs_n, Wv_n, bv_n, Wr_s, br_s,
                                 Wr_v, br_v, r48)
    return (s_out, v_out48.reshape(N, NV, 3))


# speedup vs baseline: 1.2607x; 1.2607x over previous
import functools

import jax
import jax.numpy as jnp
from jax import lax
from jax.experimental import pallas as pl
from jax.experimental.pallas import tpu as pltpu
from jax.experimental.pallas import tpu_sc as plsc

N = 10000
E = 320000
NS, NV = 128, 16
ES, EV = 32, 1

HW = 128
BE = 2000
NSC = 2
NSUB = 16
EPW = E // NSUB
CH = 80
NCH = EPW // CH
NPAD = 10240
RPW = NPAD // NSUB

_HI = lax.Precision.HIGHEST


def _mm(a, b_t, prec=lax.Precision.HIGHEST):
    return lax.dot_general(a, b_t, (((1,), (1,)), ((), ())),
                           precision=prec, preferred_element_type=jnp.float32)


def _edge_kernel(es_ref, ev_ref, ws_ref, bs_ref, wv_ref, bv_ref, r_ref,
                 out_ref):
    es = es_ref[...]
    hi3 = lax.Precision.DEFAULT
    s_lin = _mm(es, ws_ref[...], hi3) + bs_ref[...]
    s_out = s_lin * jax.nn.sigmoid(s_lin)
    v_lin = _mm(ev_ref[...], wv_ref[...], hi3) + bv_ref[...]
    gate = jax.nn.sigmoid(s_out[:, :NV])
    gate48 = _mm(gate, r_ref[...], hi3)
    v_out = v_lin * gate48
    ones = jnp.ones((es.shape[0], 1), jnp.float32)
    pad = jnp.zeros((es.shape[0], HW - 3 * NV - 1), jnp.float32)
    out_ref[0] = s_out
    out_ref[1] = jnp.concatenate([v_out, ones, pad], axis=1)


def _edge_stage(edge_s, edge_v3, Ws_e, bs_e, Wv_e, bv_e, r48):
    grid = (E // BE,)
    return pl.pallas_call(
        _edge_kernel,
        grid=grid,
        in_specs=[
            pl.BlockSpec((BE, ES), lambda i: (i, 0)),
            pl.BlockSpec((BE, 3), lambda i: (i, 0)),
            pl.BlockSpec((NS, ES), lambda i: (0, 0)),
            pl.BlockSpec((1, NS), lambda i: (0, 0)),
            pl.BlockSpec((3 * NV, 3), lambda i: (0, 0)),
            pl.BlockSpec((1, 3 * NV), lambda i: (0, 0)),
            pl.BlockSpec((3 * NV, NV), lambda i: (0, 0)),
        ],
        out_specs=pl.BlockSpec((NSC, BE, HW), lambda i: (0, i, 0)),
        out_shape=jax.ShapeDtypeStruct((NSC, E, HW), jnp.float32),
    )(edge_s, edge_v3, Ws_e, bs_e.reshape(1, NS), Wv_e,
      bv_e.reshape(1, 3 * NV), r48)


def _scatter_body(msg_hbm, dst_hbm, out_hbm, idx2, msg2, zero_v, acc_sh,
                  sem0, sem1):
    c = lax.axis_index("c")
    s = lax.axis_index("s")
    ebase = s * EPW
    sems = (sem0, sem1)

    def fire(slot, e0):
        e0 = pl.multiple_of(e0, 8)
        pltpu.async_copy(dst_hbm.at[pl.ds(e0, CH)], idx2.at[slot], sems[slot])
        pltpu.async_copy(msg_hbm.at[c, pl.ds(e0, CH)], msg2.at[slot],
                         sems[slot])

    def drain(slot):
        pltpu.make_async_copy(dst_hbm.at[pl.ds(0, CH)], idx2.at[slot],
                              sems[slot]).wait()
        pltpu.make_async_copy(msg_hbm.at[0, pl.ds(0, CH)], msg2.at[slot],
                              sems[slot]).wait()

    def scat(slot):
        pltpu.sync_copy(msg2.at[slot], acc_sh.at[idx2.at[slot]], add=True)

    fire(0, ebase)

    def zrow(r, carry):
        for g in range(HW // 16):
            zero_v[r, pl.ds(g * 16, 16)] = jnp.zeros((16,), jnp.float32)
        return carry
    lax.fori_loop(0, zero_v.shape[0], zrow, 0)
    zr = zero_v.shape[0]
    for i in range(RPW // zr):
        pltpu.sync_copy(zero_v, acc_sh.at[pl.ds(s * RPW + i * zr, zr)])
    plsc.subcore_barrier()

    def pair(i2, carry):
        base2 = ebase + i2 * (2 * CH)
        drain(0)
        fire(1, base2 + CH)
        scat(0)
        drain(1)

        @pl.when(i2 + 1 < NCH // 2)
        def _():
            fire(0, base2 + 2 * CH)
        scat(1)
        return carry
    lax.fori_loop(0, NCH // 2, pair, 0)
    plsc.subcore_barrier()

    pltpu.sync_copy(acc_sh.at[pl.ds(s * RPW, RPW)],
                    out_hbm.at[c, pl.ds(s * RPW, RPW)])


def _scatter_stage(msg, dst):
    mesh = plsc.VectorSubcoreMesh(core_axis_name="c", subcore_axis_name="s")
    f = pl.kernel(
        _scatter_body,
        out_type=jax.ShapeDtypeStruct((NSC, NPAD, HW), jnp.float32),
        mesh=mesh,
        scratch_types=[
            pltpu.VMEM((2, CH), jnp.int32),
            pltpu.VMEM((2, CH, HW), jnp.float32),
            pltpu.VMEM((128, HW), jnp.float32),
            pltpu.VMEM_SHARED((NPAD, HW), jnp.float32),
            pltpu.SemaphoreType.DMA,
            pltpu.SemaphoreType.DMA,
        ],
    )
    return f(msg, dst)


def _node_kernel(p0_ref, p1_ref, ns_ref, nv_ref, lng_ref, lnb_ref,
                 wsn_ref, bsn_ref, wvn_ref, bvn_ref,
                 wrs_ref, brs_ref, wrv_ref, brv_ref, r_ref,
                 so_ref, vo_ref):
    p1 = p1_ref[...]
    cnt = p1[:, 3 * NV:3 * NV + 1]
    denom = jnp.maximum(cnt, 1.0)
    s_agg = p0_ref[...] / denom
    v_agg = p1[:, :3 * NV] / denom

    ns = ns_ref[...]
    mu = jnp.mean(ns, axis=1, keepdims=True)
    var = jnp.mean((ns - mu) ** 2, axis=1, keepdims=True)
    s = (ns - mu) / jnp.sqrt(var + 1e-5) * lng_ref[...] + lnb_ref[...]

    nv = nv_ref[...]
    vn = jnp.sqrt(jnp.sum(nv * nv, axis=1, keepdims=True) / NV + 1e-8)
    v = nv / vn

    hs_in = s + s_agg
    hv_in = v + v_agg
    s_lin = _mm(hs_in, wsn_ref[...]) + bsn_ref[...]
    h_s = s_lin * jax.nn.sigmoid(s_lin)
    v_lin = _mm(hv_in, wvn_ref[...]) + bvn_ref[...]
    gate48 = _mm(jax.nn.sigmoid(h_s[:, :NV]), r_ref[...])
    h_v = v_lin * gate48

    so_ref[...] = h_s + _mm(s, wrs_ref[...]) + brs_ref[...]
    vo_ref[...] = h_v + _mm(v, wrv_ref[...]) + brv_ref[...]


BN = 2000


def _node_stage(partials, node_s, node_v48, ln_g, ln_b, Ws_n, bs_n, Wv_n,
                bv_n, Wr_s, br_s, Wr_v, br_v, r48):
    blk = lambda shape: pl.BlockSpec(shape, lambda i: (i,) + (0,) * (len(shape) - 1))
    fix = lambda shape: pl.BlockSpec(shape, lambda i: (0,) * len(shape))
    return pl.pallas_call(
        _node_kernel,
        grid=(N // BN,),
        in_specs=[
            blk((BN, HW)), blk((BN, HW)), blk((BN, NS)), blk((BN, 3 * NV)),
            fix((1, NS)), fix((1, NS)),
            fix((NS, NS)), fix((1, NS)),
            fix((3 * NV, 3 * NV)), fix((1, 3 * NV)),
            fix((NS, NS)), fix((1, NS)),
            fix((3 * NV, 3 * NV)), fix((1, 3 * NV)),
            fix((3 * NV, NV)),
        ],
        out_specs=[blk((BN, NS)), blk((BN, 3 * NV))],
        out_shape=[jax.ShapeDtypeStruct((N, NS), jnp.float32),
                   jax.ShapeDtypeStruct((N, 3 * NV), jnp.float32)],
    )(partials[0], partials[1], node_s, node_v48,
      ln_g.reshape(1, NS), ln_b.reshape(1, NS),
      Ws_n, bs_n.reshape(1, NS), Wv_n, bv_n.reshape(1, 3 * NV),
      Wr_s, br_s.reshape(1, NS), Wr_v, br_v.reshape(1, 3 * NV), r48)


def kernel(node_s, node_v, edge_s, edge_v, ln_g, ln_b, Ws_e, bs_e, Wv_e,
           bv_e, Ws_n, bs_n, Wv_n, bv_n, Wr_s, br_s, Wr_v, br_v, edge_index):
    edge_v3 = edge_v.reshape(E, 3 * EV)
    node_v48 = node_v.reshape(N, 3 * NV)
    dst = edge_index[1].astype(jnp.int32)
    r48 = jnp.kron(jnp.eye(NV, dtype=jnp.float32),
                   jnp.ones((3, 1), jnp.float32))

    msg = _edge_stage(edge_s, edge_v3, Ws_e, bs_e, Wv_e, bv_e, r48)
    partials = _scatter_stage(msg, dst)[:, :N, :]
    s_out, v_out48 = _node_stage(partials, node_s, node_v48, ln_g, ln_b,
                                 Ws_n, bs_n, Wv_n, bv_n, Wr_s, br_s,
                                 Wr_v, br_v, r48)
    return (s_out, v_out48.reshape(N, NV, 3))


# speedup vs baseline: 1.3560x; 1.0756x over previous
import functools

import jax
import jax.numpy as jnp
from jax import lax
from jax.experimental import pallas as pl
from jax.experimental.pallas import tpu as pltpu
from jax.experimental.pallas import tpu_sc as plsc

N = 10000
E = 320000
NS, NV = 128, 16
ES, EV = 32, 1

HW = 128
BE = 2000
NSC = 2
NSUB = 16
EPW = E // NSUB
CH = 80
NCH = EPW // CH
NPAD = 10240
RPW = NPAD // NSUB

_HI = lax.Precision.HIGHEST


def _mm(a, b_t, prec=lax.Precision.HIGHEST):
    return lax.dot_general(a, b_t, (((1,), (1,)), ((), ())),
                           precision=prec, preferred_element_type=jnp.float32)


def _edge_kernel(es_ref, ev_ref, ws_ref, bs_ref, wv_ref, bv_ref, r_ref,
                 out_ref):
    es = es_ref[...]
    hi3 = lax.Precision.DEFAULT
    s_lin = _mm(es, ws_ref[...], hi3) + bs_ref[...]
    s_out = s_lin * jax.nn.sigmoid(s_lin)
    v_lin = _mm(ev_ref[...], wv_ref[...], hi3) + bv_ref[...]
    gate = jax.nn.sigmoid(s_out[:, :NV])
    gate48 = _mm(gate, r_ref[...], hi3)
    v_out = v_lin * gate48
    ones = jnp.ones((es.shape[0], 1), jnp.float32)
    pad = jnp.zeros((es.shape[0], HW - 3 * NV - 1), jnp.float32)
    out_ref[0] = s_out
    out_ref[1] = jnp.concatenate([v_out, ones, pad], axis=1)


def _edge_stage(edge_s, edge_v3, Ws_e, bs_e, Wv_e, bv_e, r48, phase, ne):
    off = phase * (ne // BE)
    return pl.pallas_call(
        _edge_kernel,
        grid=(ne // BE,),
        in_specs=[
            pl.BlockSpec((BE, ES), lambda i: (i + off, 0)),
            pl.BlockSpec((BE, 3), lambda i: (i + off, 0)),
            pl.BlockSpec((NS, ES), lambda i: (0, 0)),
            pl.BlockSpec((1, NS), lambda i: (0, 0)),
            pl.BlockSpec((3 * NV, 3), lambda i: (0, 0)),
            pl.BlockSpec((1, 3 * NV), lambda i: (0, 0)),
            pl.BlockSpec((3 * NV, NV), lambda i: (0, 0)),
        ],
        out_specs=pl.BlockSpec((NSC, BE, HW), lambda i: (0, i, 0)),
        out_shape=jax.ShapeDtypeStruct((NSC, ne, HW), jnp.float32),
    )(edge_s, edge_v3, Ws_e, bs_e.reshape(1, NS), Wv_e,
      bv_e.reshape(1, 3 * NV), r48)


def _make_scatter_body(phase, ne):
    epw = ne // NSUB
    nch = epw // CH

    def _scatter_body(msg_hbm, dst_hbm, out_hbm, idx2, msg2, zero_v, acc_sh,
                      sem0, sem1):
        c = lax.axis_index("c")
        s = lax.axis_index("s")
        lbase = s * epw
        gbase = phase * ne + lbase
        sems = (sem0, sem1)

        def fire(slot, eloc):
            el = pl.multiple_of(eloc, 8)
            eg = pl.multiple_of(eloc + phase * ne, 8)
            pltpu.async_copy(dst_hbm.at[pl.ds(eg, CH)], idx2.at[slot],
                             sems[slot])
            pltpu.async_copy(msg_hbm.at[c, pl.ds(el, CH)], msg2.at[slot],
                             sems[slot])

        def drain(slot):
            pltpu.make_async_copy(dst_hbm.at[pl.ds(0, CH)], idx2.at[slot],
                                  sems[slot]).wait()
            pltpu.make_async_copy(msg_hbm.at[0, pl.ds(0, CH)], msg2.at[slot],
                                  sems[slot]).wait()

        def scat(slot):
            pltpu.sync_copy(msg2.at[slot], acc_sh.at[idx2.at[slot]], add=True)

        fire(0, lbase)

        def zrow(r, carry):
            for g in range(HW // 16):
                zero_v[r, pl.ds(g * 16, 16)] = jnp.zeros((16,), jnp.float32)
            return carry
        lax.fori_loop(0, zero_v.shape[0], zrow, 0)
        zr = zero_v.shape[0]
        for i in range(RPW // zr):
            pltpu.sync_copy(zero_v, acc_sh.at[pl.ds(s * RPW + i * zr, zr)])
        plsc.subcore_barrier()

        def pair(i2, carry):
            base2 = lbase + i2 * (2 * CH)
            drain(0)
            fire(1, base2 + CH)
            scat(0)
            drain(1)

            @pl.when(2 * i2 + 2 < nch)
            def _():
                fire(0, base2 + 2 * CH)
            scat(1)
            return carry
        lax.fori_loop(0, nch // 2, pair, 0)
        if nch % 2:
            drain(0)
            scat(0)
        plsc.subcore_barrier()

        pltpu.sync_copy(acc_sh.at[pl.ds(s * RPW, RPW)],
                        out_hbm.at[c, pl.ds(s * RPW, RPW)])

    return _scatter_body


def _scatter_stage(msg, dst, phase, ne):
    mesh = plsc.VectorSubcoreMesh(core_axis_name="c", subcore_axis_name="s")
    f = pl.kernel(
        _make_scatter_body(phase, ne),
        out_type=jax.ShapeDtypeStruct((NSC, NPAD, HW), jnp.float32),
        mesh=mesh,
        scratch_types=[
            pltpu.VMEM((2, CH), jnp.int32),
            pltpu.VMEM((2, CH, HW), jnp.float32),
            pltpu.VMEM((128, HW), jnp.float32),
            pltpu.VMEM_SHARED((NPAD, HW), jnp.float32),
            pltpu.SemaphoreType.DMA,
            pltpu.SemaphoreType.DMA,
        ],
    )
    return f(msg, dst)


def _node_kernel(p0a_ref, p1a_ref, p0b_ref, p1b_ref, ns_ref, nv_ref,
                 lng_ref, lnb_ref,
                 wsn_ref, bsn_ref, wvn_ref, bvn_ref,
                 wrs_ref, brs_ref, wrv_ref, brv_ref, r_ref,
                 so_ref, vo_ref):
    p1 = p1a_ref[...] + p1b_ref[...]
    cnt = p1[:, 3 * NV:3 * NV + 1]
    denom = jnp.maximum(cnt, 1.0)
    s_agg = (p0a_ref[...] + p0b_ref[...]) / denom
    v_agg = p1[:, :3 * NV] / denom

    ns = ns_ref[...]
    mu = jnp.mean(ns, axis=1, keepdims=True)
    var = jnp.mean((ns - mu) ** 2, axis=1, keepdims=True)
    s = (ns - mu) / jnp.sqrt(var + 1e-5) * lng_ref[...] + lnb_ref[...]

    nv = nv_ref[...]
    vn = jnp.sqrt(jnp.sum(nv * nv, axis=1, keepdims=True) / NV + 1e-8)
    v = nv / vn

    hs_in = s + s_agg
    hv_in = v + v_agg
    s_lin = _mm(hs_in, wsn_ref[...]) + bsn_ref[...]
    h_s = s_lin * jax.nn.sigmoid(s_lin)
    v_lin = _mm(hv_in, wvn_ref[...]) + bvn_ref[...]
    gate48 = _mm(jax.nn.sigmoid(h_s[:, :NV]), r_ref[...])
    h_v = v_lin * gate48

    so_ref[...] = h_s + _mm(s, wrs_ref[...]) + brs_ref[...]
    vo_ref[...] = h_v + _mm(v, wrv_ref[...]) + brv_ref[...]


BN = 2000


def _node_stage(pa, pb, node_s, node_v48, ln_g, ln_b, Ws_n, bs_n, Wv_n,
                bv_n, Wr_s, br_s, Wr_v, br_v, r48):
    blk = lambda shape: pl.BlockSpec(shape, lambda i: (i,) + (0,) * (len(shape) - 1))
    fix = lambda shape: pl.BlockSpec(shape, lambda i: (0,) * len(shape))
    return pl.pallas_call(
        _node_kernel,
        grid=(N // BN,),
        in_specs=[
            blk((BN, HW)), blk((BN, HW)), blk((BN, HW)), blk((BN, HW)),
            blk((BN, NS)), blk((BN, 3 * NV)),
            fix((1, NS)), fix((1, NS)),
            fix((NS, NS)), fix((1, NS)),
            fix((3 * NV, 3 * NV)), fix((1, 3 * NV)),
            fix((NS, NS)), fix((1, NS)),
            fix((3 * NV, 3 * NV)), fix((1, 3 * NV)),
            fix((3 * NV, NV)),
        ],
        out_specs=[blk((BN, NS)), blk((BN, 3 * NV))],
        out_shape=[jax.ShapeDtypeStruct((N, NS), jnp.float32),
                   jax.ShapeDtypeStruct((N, 3 * NV), jnp.float32)],
    )(pa[0], pa[1], pb[0], pb[1], node_s, node_v48,
      ln_g.reshape(1, NS), ln_b.reshape(1, NS),
      Ws_n, bs_n.reshape(1, NS), Wv_n, bv_n.reshape(1, 3 * NV),
      Wr_s, br_s.reshape(1, NS), Wr_v, br_v.reshape(1, 3 * NV), r48)


def kernel(node_s, node_v, edge_s, edge_v, ln_g, ln_b, Ws_e, bs_e, Wv_e,
           bv_e, Ws_n, bs_n, Wv_n, bv_n, Wr_s, br_s, Wr_v, br_v, edge_index):
    edge_v3 = edge_v.reshape(E, 3 * EV)
    node_v48 = node_v.reshape(N, 3 * NV)
    dst = edge_index[1].astype(jnp.int32)
    r48 = jnp.kron(jnp.eye(NV, dtype=jnp.float32),
                   jnp.ones((3, 1), jnp.float32))

    ne = E // 2
    msg0 = _edge_stage(edge_s, edge_v3, Ws_e, bs_e, Wv_e, bv_e, r48, 0, ne)
    pa = _scatter_stage(msg0, dst, 0, ne)[:, :N, :]
    msg1 = _edge_stage(edge_s, edge_v3, Ws_e, bs_e, Wv_e, bv_e, r48, 1, ne)
    pb = _scatter_stage(msg1, dst, 1, ne)[:, :N, :]
    s_out, v_out48 = _node_stage(pa, pb, node_s, node_v48, ln_g, ln_b,
                                 Ws_n, bs_n, Wv_n, bv_n, Wr_s, br_s,
                                 Wr_v, br_v, r48)
    return (s_out, v_out48.reshape(N, NV, 3))
